# baseline (speedup 1.0000x reference)
import jax
import jax.numpy as jnp
from jax.experimental import pallas as pl
from jax.experimental.pallas import tpu as pltpu

_MAX_K = 128
_SEQ = 2048
_D = 64
_T_ROWS = 2 * _MAX_K + 1
_ROWS128 = _SEQ * _D // 128
_HALF = _SEQ // 2


def _band_body(w_ref, out_ref, a_ref, b_ref, sem):
    w = w_ref[...]
    c00 = jnp.concatenate([w[0:1, :], w[0:1, :]], axis=1)
    czz = jnp.concatenate([w[_T_ROWS - 1:, :], w[_T_ROWS - 1:, :]], axis=1)
    k_i = jax.lax.broadcasted_iota(jnp.int32, (128, 256), 0)
    r_i = jax.lax.broadcasted_iota(jnp.int32, (128, 256), 1)
    p_even = (r_i == 2 * k_i).astype(jnp.float32)
    p_odd = (r_i == 2 * k_i + 1).astype(jnp.float32)
    dot = lambda p, m: jax.lax.dot_general(
        p, m, (((1,), (0,)), ((), ())), preferred_element_type=jnp.float32)
    w1 = w[1:257, :]
    w0 = w[0:256, :]
    a_ref[0:960, :] = jnp.broadcast_to(c00, (960, 128))
    a_ref[960:1088, :] = jnp.concatenate([dot(p_even, w1), dot(p_odd, w1)],
                                         axis=1)
    a_ref[1088:2048, :] = jnp.broadcast_to(czz, (960, 128))
    b_ref[0:959, :] = jnp.broadcast_to(c00, (959, 128))
    b_ref[959:1087, :] = jnp.concatenate([dot(p_even, w0), dot(p_odd, w0)],
                                         axis=1)
    b_ref[1087:2048, :] = jnp.broadcast_to(czz, (961, 128))

    def issue(s, _):
        src_a = a_ref.at[pl.ds(_HALF - 1 - s, _ROWS128), :]
        src_b = b_ref.at[pl.ds(_HALF - 1 - s, _ROWS128), :]
        pltpu.make_async_copy(src_a, out_ref.at[2 * s + 1], sem).start()
        pltpu.make_async_copy(src_b, out_ref.at[2 * s], sem).start()
        return 0

    jax.lax.fori_loop(0, _HALF, issue, 0)

    def drain(s, _):
        pltpu.make_async_copy(a_ref.at[pl.ds(0, _ROWS128), :],
                              out_ref.at[0], sem).wait()
        return 0

    jax.lax.fori_loop(0, _SEQ, drain, 0)


def kernel(seq_len, emb_weight):
    del seq_len
    out = pl.pallas_call(
        _band_body,
        grid=(1,),
        in_specs=[pl.BlockSpec((_T_ROWS, _D), lambda i: (0, 0))],
        out_specs=pl.BlockSpec(memory_space=pltpu.MemorySpace.HBM),
        out_shape=jax.ShapeDtypeStruct((_SEQ, _ROWS128, 128), jnp.float32),
        scratch_shapes=[pltpu.VMEM((_SEQ, 128), jnp.float32),
                        pltpu.VMEM((_SEQ, 128), jnp.float32),
                        pltpu.SemaphoreType.DMA],
    )(emb_weight)
    return out.reshape(_SEQ, _SEQ, _D)


# speedup vs baseline: 1.0003x; 1.0003x over previous
import jax
import jax.numpy as jnp
from jax.experimental import pallas as pl
from jax.experimental.pallas import tpu as pltpu

_MAX_K = 128
_SEQ = 2048
_D = 64
_T_ROWS = 2 * _MAX_K + 1
_ROWS128 = _SEQ * _D // 128
_HALF = _SEQ // 2
_NSEM = 8


def _band_body(w_ref, out_ref, a_ref, b_ref, sem):
    w = w_ref[...]
    c00 = jnp.concatenate([w[0:1, :], w[0:1, :]], axis=1)
    czz = jnp.concatenate([w[_T_ROWS - 1:, :], w[_T_ROWS - 1:, :]], axis=1)
    k_i = jax.lax.broadcasted_iota(jnp.int32, (128, 256), 0)
    r_i = jax.lax.broadcasted_iota(jnp.int32, (128, 256), 1)
    p_even = (r_i == 2 * k_i).astype(jnp.float32)
    p_odd = (r_i == 2 * k_i + 1).astype(jnp.float32)
    dot = lambda p, m: jax.lax.dot_general(
        p, m, (((1,), (0,)), ((), ())), preferred_element_type=jnp.float32)
    w1 = w[1:257, :]
    w0 = w[0:256, :]
    a_ref[0:960, :] = jnp.broadcast_to(c00, (960, 128))
    a_ref[960:1088, :] = jnp.concatenate([dot(p_even, w1), dot(p_odd, w1)],
                                         axis=1)
    a_ref[1088:2048, :] = jnp.broadcast_to(czz, (960, 128))
    b_ref[0:959, :] = jnp.broadcast_to(c00, (959, 128))
    b_ref[959:1087, :] = jnp.concatenate([dot(p_even, w0), dot(p_odd, w0)],
                                         axis=1)
    b_ref[1087:2048, :] = jnp.broadcast_to(czz, (961, 128))

    def issue(s, _):
        src_a = a_ref.at[pl.ds(_HALF - 1 - s, _ROWS128), :]
        src_b = b_ref.at[pl.ds(_HALF - 1 - s, _ROWS128), :]
        k = jax.lax.rem(s, _NSEM)
        pltpu.make_async_copy(src_a, out_ref.at[2 * s + 1], sem.at[k]).start()
        pltpu.make_async_copy(src_b, out_ref.at[2 * s], sem.at[k]).start()
        return 0

    jax.lax.fori_loop(0, _HALF, issue, 0)

    def drain(s, _):
        k = jax.lax.rem(s, _NSEM)
        pltpu.make_async_copy(a_ref.at[pl.ds(0, _ROWS128), :],
                              out_ref.at[0], sem.at[k]).wait()
        return 0

    jax.lax.fori_loop(0, _HALF, drain, 0)
    jax.lax.fori_loop(0, _HALF, drain, 0)


def kernel(seq_len, emb_weight):
    del seq_len
    out = pl.pallas_call(
        _band_body,
        grid=(1,),
        in_specs=[pl.BlockSpec((_T_ROWS, _D), lambda i: (0, 0))],
        out_specs=pl.BlockSpec(memory_space=pltpu.MemorySpace.HBM),
        out_shape=jax.ShapeDtypeStruct((_SEQ, _ROWS128, 128), jnp.float32),
        scratch_shapes=[pltpu.VMEM((_SEQ, 128), jnp.float32),
                        pltpu.VMEM((_SEQ, 128), jnp.float32),
                        pltpu.SemaphoreType.DMA((_NSEM,))],
    )(emb_weight)
    return out.reshape(_SEQ, _SEQ, _D)
